# baseline (speedup 1.0000x reference)
import jax
import jax.numpy as jnp
from jax.experimental import pallas as pl
from jax.experimental.pallas import tpu as pltpu

_B, _E, _A = 128, 4, 100000
_AC = 4096
_NC = 25
_AC2 = 2048
_G = _AC // 8
_IMAX = 2147483647


def _stats_body(qt_ref, q_ref, c_ref, idx_ref, accM, accS, accI):
    i = pl.program_id(0)

    @pl.when(i == 0)
    def _init():
        accM[...] = jnp.full((8, _B), -jnp.inf, jnp.float32)
        accS[...] = jnp.zeros((8, _B), jnp.float32)
        accI[...] = jnp.full((8, _B), _IMAX, jnp.int32)

    q = jnp.min(qt_ref[...], axis=0)
    q_ref[...] = q
    ids = (jax.lax.broadcasted_iota(jnp.int32, (_AC, _B), 0)
           + i * _AC)
    qv = jnp.where(ids < _A, q, -jnp.inf)
    q3 = qv.reshape(_G, 8, _B)
    i3 = ids.reshape(_G, 8, _B)
    m_c = jnp.max(q3, axis=0)
    i_c = jnp.min(jnp.where(q3 == m_c[None], i3, jnp.int32(_IMAX)), axis=0)
    m_old = accM[...]
    m_run = jnp.maximum(m_old, m_c)
    s_c = jnp.sum(jnp.exp(q3 - m_run[None]), axis=0)
    accS[...] = accS[...] * jnp.exp(m_old - m_run) + s_c
    accI[...] = jnp.where(m_c > m_old, i_c, accI[...])
    accM[...] = m_run

    @pl.when(i == _NC - 1)
    def _fin():
        M, S, I = accM[...], accS[...], accI[...]
        m_g = jnp.max(M, axis=0, keepdims=True)
        lse = jnp.log(jnp.sum(S * jnp.exp(M - m_g), axis=0, keepdims=True))
        best = jnp.min(jnp.where(M == m_g, I, jnp.int32(_IMAX)), axis=0, keepdims=True)
        c_ref[...] = jnp.broadcast_to(m_g + lse, (8, _B))
        idx_ref[...] = jnp.broadcast_to(best, (8, _B))


def _write_body(q_ref, c_ref, lp_ref):
    lp_ref[...] = q_ref[...] - c_ref[0:1, :]


def kernel(Qs):
    qt = jnp.transpose(Qs, (1, 2, 0))
    q, c, idx = pl.pallas_call(
        _stats_body,
        grid=(_NC,),
        in_specs=[pl.BlockSpec((_E, _AC, _B), lambda i: (0, i, 0))],
        out_specs=[
            pl.BlockSpec((_AC, _B), lambda i: (i, 0)),
            pl.BlockSpec((8, _B), lambda i: (0, 0)),
            pl.BlockSpec((8, _B), lambda i: (0, 0)),
        ],
        out_shape=[
            jax.ShapeDtypeStruct((_A, _B), jnp.float32),
            jax.ShapeDtypeStruct((8, _B), jnp.float32),
            jax.ShapeDtypeStruct((8, _B), jnp.int32),
        ],
        scratch_shapes=[
            pltpu.VMEM((8, _B), jnp.float32),
            pltpu.VMEM((8, _B), jnp.float32),
            pltpu.VMEM((8, _B), jnp.int32),
        ],
    )(qt)
    lp_t = pl.pallas_call(
        _write_body,
        grid=(_A // _AC2 + 1,),
        in_specs=[
            pl.BlockSpec((_AC2, _B), lambda i: (i, 0)),
            pl.BlockSpec((8, _B), lambda i: (0, 0)),
        ],
        out_specs=pl.BlockSpec((_AC2, _B), lambda i: (i, 0)),
        out_shape=jax.ShapeDtypeStruct((_A, _B), jnp.float32),
    )(q, c)
    return jnp.transpose(lp_t), idx[0]


# speedup vs baseline: 1.1625x; 1.1625x over previous
import jax
import jax.numpy as jnp
from jax.experimental import pallas as pl
from jax.experimental.pallas import tpu as pltpu

_B, _E, _A = 128, 4, 100000
_AC = 4096
_NC = 25
_AC2 = 8192
_G = _AC // 8
_IMAX = 2147483647


def _stats_body(qt_ref, q_ref, c_ref, idx_ref, accM, accS, accI):
    i = pl.program_id(0)

    @pl.when(i == 0)
    def _init():
        accM[...] = jnp.full((8, _B), -jnp.inf, jnp.float32)
        accS[...] = jnp.zeros((8, _B), jnp.float32)
        accI[...] = jnp.full((8, _B), _IMAX, jnp.int32)

    q = jnp.min(qt_ref[...], axis=0)
    q_ref[...] = q
    ids = (jax.lax.broadcasted_iota(jnp.int32, (_AC, _B), 0)
           + i * _AC)
    qv = jnp.where(ids < _A, q, -jnp.inf)
    q3 = qv.reshape(_G, 8, _B)
    i3 = ids.reshape(_G, 8, _B)
    m_c = jnp.max(q3, axis=0)
    i_c = jnp.min(jnp.where(q3 == m_c[None], i3, jnp.int32(_IMAX)), axis=0)
    m_old = accM[...]
    m_run = jnp.maximum(m_old, m_c)
    s_c = jnp.sum(jnp.exp(q3 - m_run[None]), axis=0)
    accS[...] = accS[...] * jnp.exp(m_old - m_run) + s_c
    accI[...] = jnp.where(m_c > m_old, i_c, accI[...])
    accM[...] = m_run

    @pl.when(i == _NC - 1)
    def _fin():
        M, S, I = accM[...], accS[...], accI[...]
        m_g = jnp.max(M, axis=0, keepdims=True)
        lse = jnp.log(jnp.sum(S * jnp.exp(M - m_g), axis=0, keepdims=True))
        best = jnp.min(jnp.where(M == m_g, I, jnp.int32(_IMAX)), axis=0, keepdims=True)
        c_ref[...] = jnp.broadcast_to(m_g + lse, (8, _B))
        idx_ref[...] = jnp.broadcast_to(best, (8, _B))


def _write_body(q_ref, c_ref, lp_ref):
    lp_ref[...] = q_ref[...] - c_ref[0:1, :]


def kernel(Qs):
    qt = jnp.transpose(Qs, (1, 2, 0))
    q, c, idx = pl.pallas_call(
        _stats_body,
        grid=(_NC,),
        in_specs=[pl.BlockSpec((_E, _AC, _B), lambda i: (0, i, 0))],
        out_specs=[
            pl.BlockSpec((_AC, _B), lambda i: (i, 0)),
            pl.BlockSpec((8, _B), lambda i: (0, 0)),
            pl.BlockSpec((8, _B), lambda i: (0, 0)),
        ],
        out_shape=[
            jax.ShapeDtypeStruct((_A, _B), jnp.float32),
            jax.ShapeDtypeStruct((8, _B), jnp.float32),
            jax.ShapeDtypeStruct((8, _B), jnp.int32),
        ],
        scratch_shapes=[
            pltpu.VMEM((8, _B), jnp.float32),
            pltpu.VMEM((8, _B), jnp.float32),
            pltpu.VMEM((8, _B), jnp.int32),
        ],
    )(qt)
    lp_t = pl.pallas_call(
        _write_body,
        grid=(_A // _AC2 + 1,),
        in_specs=[
            pl.BlockSpec((_AC2, _B), lambda i: (i, 0)),
            pl.BlockSpec((8, _B), lambda i: (0, 0)),
        ],
        out_specs=pl.BlockSpec((_AC2, _B), lambda i: (i, 0)),
        out_shape=jax.ShapeDtypeStruct((_A, _B), jnp.float32),
    )(q, c)
    return jnp.transpose(lp_t), idx[0]


# speedup vs baseline: 1.1688x; 1.0054x over previous
import jax
import jax.numpy as jnp
from jax.experimental import pallas as pl
from jax.experimental.pallas import tpu as pltpu

_B, _E, _A = 128, 4, 100000
_AC = 4096
_NC = 25
_AC2 = 16384
_G = _AC // 8
_IMAX = 2147483647


def _stats_body(qt_ref, q_ref, c_ref, idx_ref, accM, accS, accI):
    i = pl.program_id(0)

    @pl.when(i == 0)
    def _init():
        accM[...] = jnp.full((8, _B), -jnp.inf, jnp.float32)
        accS[...] = jnp.zeros((8, _B), jnp.float32)
        accI[...] = jnp.full((8, _B), _IMAX, jnp.int32)

    q = jnp.min(qt_ref[...], axis=0)
    q_ref[...] = q
    ids = (jax.lax.broadcasted_iota(jnp.int32, (_AC, _B), 0)
           + i * _AC)
    qv = jnp.where(ids < _A, q, -jnp.inf)
    q3 = qv.reshape(_G, 8, _B)
    i3 = ids.reshape(_G, 8, _B)
    m_c = jnp.max(q3, axis=0)
    i_c = jnp.min(jnp.where(q3 == m_c[None], i3, jnp.int32(_IMAX)), axis=0)
    m_old = accM[...]
    m_run = jnp.maximum(m_old, m_c)
    s_c = jnp.sum(jnp.exp(q3 - m_run[None]), axis=0)
    accS[...] = accS[...] * jnp.exp(m_old - m_run) + s_c
    accI[...] = jnp.where(m_c > m_old, i_c, accI[...])
    accM[...] = m_run

    @pl.when(i == _NC - 1)
    def _fin():
        M, S, I = accM[...], accS[...], accI[...]
        m_g = jnp.max(M, axis=0, keepdims=True)
        lse = jnp.log(jnp.sum(S * jnp.exp(M - m_g), axis=0, keepdims=True))
        best = jnp.min(jnp.where(M == m_g, I, jnp.int32(_IMAX)), axis=0, keepdims=True)
        c_ref[...] = jnp.broadcast_to(m_g + lse, (8, _B))
        idx_ref[...] = jnp.broadcast_to(best, (8, _B))


def _write_body(q_ref, c_ref, lp_ref):
    lp_ref[...] = q_ref[...] - c_ref[0:1, :]


def kernel(Qs):
    qt = jnp.transpose(Qs, (1, 2, 0))
    q, c, idx = pl.pallas_call(
        _stats_body,
        grid=(_NC,),
        in_specs=[pl.BlockSpec((_E, _AC, _B), lambda i: (0, i, 0))],
        out_specs=[
            pl.BlockSpec((_AC, _B), lambda i: (i, 0)),
            pl.BlockSpec((8, _B), lambda i: (0, 0)),
            pl.BlockSpec((8, _B), lambda i: (0, 0)),
        ],
        out_shape=[
            jax.ShapeDtypeStruct((_A, _B), jnp.float32),
            jax.ShapeDtypeStruct((8, _B), jnp.float32),
            jax.ShapeDtypeStruct((8, _B), jnp.int32),
        ],
        scratch_shapes=[
            pltpu.VMEM((8, _B), jnp.float32),
            pltpu.VMEM((8, _B), jnp.float32),
            pltpu.VMEM((8, _B), jnp.int32),
        ],
    )(qt)
    lp_t = pl.pallas_call(
        _write_body,
        grid=(_A // _AC2 + 1,),
        in_specs=[
            pl.BlockSpec((_AC2, _B), lambda i: (i, 0)),
            pl.BlockSpec((8, _B), lambda i: (0, 0)),
        ],
        out_specs=pl.BlockSpec((_AC2, _B), lambda i: (i, 0)),
        out_shape=jax.ShapeDtypeStruct((_A, _B), jnp.float32),
    )(q, c)
    return jnp.transpose(lp_t), idx[0]
